# Initial kernel scaffold; baseline (speedup 1.0000x reference)
import jax
import jax.numpy as jnp
from jax.experimental import pallas as pl


def kernel(node_features, edge_index, W, a_w):
    raise NotImplementedError("write your pallas kernel here")


# speedup vs baseline: 3.7735x; 3.7735x over previous
import functools

import jax
import jax.numpy as jnp
from jax import lax
from jax.experimental import pallas as pl
from jax.experimental.pallas import tpu as pltpu
from jax.experimental.pallas import tpu_sc as plsc

N = 10000
E = 160000
DIN = 256
DOUT = 256
ALPHA = 0.2

NPAD = 10240
EPAD = 163840
CHUNKS = 80
CW = 128
ROWS_PT = NPAD // 16
HALF = 128
QW = 64


def _mm_body(x_ref, w_ref, o_ref):
    o_ref[...] = jnp.dot(x_ref[...], w_ref[...],
                         preferred_element_type=jnp.float32)


def _tc_matmul(x, w_ext):
    bn = 2000
    return pl.pallas_call(
        _mm_body,
        grid=(N // bn,),
        in_specs=[
            pl.BlockSpec((bn, DIN), lambda i: (i, 0)),
            pl.BlockSpec((DIN, 384), lambda i: (0, 0)),
        ],
        out_specs=pl.BlockSpec((bn, 384), lambda i: (i, 0)),
        out_shape=jax.ShapeDtypeStruct((N, 384), jnp.float32),
    )(x, w_ext)


_sc_mesh = plsc.VectorSubcoreMesh(core_axis_name="c", subcore_axis_name="s")


@functools.partial(
    pl.kernel,
    mesh=_sc_mesh,
    compiler_params=pltpu.CompilerParams(needs_layout_passes=False,
                                         use_tc_tiling_on_sc=False),
    out_type=jax.ShapeDtypeStruct((4, NPAD, QW), jnp.float32),
    scratch_types=[
        pltpu.VMEM((CHUNKS, CW), jnp.int32),
        pltpu.VMEM((CHUNKS, CW), jnp.int32),
        pltpu.VMEM((CHUNKS, CW), jnp.int32),
        pltpu.VMEM((NPAD,), jnp.float32),
        pltpu.VMEM((NPAD,), jnp.float32),
        pltpu.VMEM((NPAD,), jnp.float32),
        pltpu.VMEM((CHUNKS, CW), jnp.float32),
        pltpu.VMEM((CW, QW), jnp.float32),
        pltpu.VMEM_SHARED((NPAD,), jnp.float32),
        pltpu.VMEM_SHARED((NPAD, QW), jnp.float32),
        pltpu.SemaphoreType.DMA,
    ],
)
def _sc_edges(hs_hbm, s1_hbm, s2_hbm, src_hbm, tgt_hbm, out_hbm,
              src_v, tgt_v, srcadj_v, s1_v, s2_v, sums_v, att_v, rows_v,
              sums_sh, acc_sh, sem):
    t = lax.axis_index("s")
    c = lax.axis_index("c")
    zero16 = jnp.zeros((16,), jnp.float32)
    iota16 = lax.iota(jnp.int32, 16)

    pltpu.sync_copy(src_hbm.at[pl.ds(t * CHUNKS, CHUNKS)], src_v)
    pltpu.sync_copy(tgt_hbm.at[pl.ds(t * CHUNKS, CHUNKS)], tgt_v)
    pltpu.sync_copy(s1_hbm, s1_v)
    pltpu.sync_copy(s2_hbm, s2_v)

    def _zsum(i, _):
        sums_v[pl.ds(i * 16, 16)] = zero16
        return 0
    lax.fori_loop(0, ROWS_PT // 16, _zsum, 0)
    pltpu.sync_copy(sums_v.at[pl.ds(0, ROWS_PT)],
                    sums_sh.at[pl.ds(t * ROWS_PT, ROWS_PT)])

    def _edges(j, _):
        for k in range(8):
            sl = pl.ds(k * 16, 16)
            sv = src_v[j, sl]
            tv = tgt_v[j, sl]
            v1 = plsc.load_gather(s1_v, [sv])
            v2 = plsc.load_gather(s2_v, [tv])
            lg = v1 + v2
            lg = jnp.where(lg > 0.0, lg, lg * ALPHA)
            ee = jnp.exp(lg)
            gid = t * (CHUNKS * CW) + j * CW + k * 16 + iota16
            ee = jnp.where(gid < E, ee, 0.0)
            att_v[j, sl] = ee
        return 0
    lax.fori_loop(0, CHUNKS, _edges, 0)

    plsc.subcore_barrier()

    def _ssum(j, _):
        pltpu.sync_copy(att_v.at[j], sums_sh.at[tgt_v.at[j]], add=True)
        return 0
    lax.fori_loop(0, CHUNKS, _ssum, 0)

    plsc.subcore_barrier()

    pltpu.sync_copy(sums_sh, sums_v)

    def _att(j, _):
        for k in range(8):
            sl = pl.ds(k * 16, 16)
            tv = tgt_v[j, sl]
            s = plsc.load_gather(sums_v, [tv])
            att_v[j, sl] = att_v[j, sl] / (s + 1e-10)
        return 0
    lax.fori_loop(0, CHUNKS, _att, 0)

    for p in range(HALF // QW):
        def _adj(j, _):
            for k in range(8):
                sl = pl.ds(k * 16, 16)
                srcadj_v[j, sl] = src_v[j, sl] + (c * 2 + p) * N
            return 0
        lax.fori_loop(0, CHUNKS, _adj, 0)

        def _zrow(r, _):
            for k in range(QW // 16):
                rows_v[r, pl.ds(k * 16, 16)] = zero16
            return 0
        lax.fori_loop(0, CW, _zrow, 0)
        for b in range(ROWS_PT // CW):
            pltpu.sync_copy(rows_v, acc_sh.at[pl.ds(t * ROWS_PT + b * CW, CW)])

        plsc.subcore_barrier()

        def _rows(j, _):
            pltpu.async_copy(hs_hbm.at[srcadj_v.at[j]], rows_v, sem).wait()

            def _scale(g, _):
                av = att_v[j, pl.ds(g * 16, 16)]
                for ri in range(16):
                    a = av[ri]
                    r = g * 16 + ri
                    for k in range(QW // 16):
                        sl = pl.ds(k * 16, 16)
                        rows_v[r, sl] = rows_v[r, sl] * a
                return 0
            lax.fori_loop(0, CW // 16, _scale, 0)
            pltpu.sync_copy(rows_v, acc_sh.at[tgt_v.at[j]], add=True)
            return 0
        lax.fori_loop(0, CHUNKS, _rows, 0)

        plsc.subcore_barrier()

        for b in range(ROWS_PT // CW):
            r0 = t * ROWS_PT + b * CW
            pltpu.sync_copy(acc_sh.at[pl.ds(r0, CW)], rows_v)

            def _elu(r, _):
                for k in range(QW // 16):
                    sl = pl.ds(k * 16, 16)
                    v = rows_v[r, sl]
                    rows_v[r, sl] = jnp.where(v > 0.0, v, jnp.exp(v) - 1.0)
                return 0
            lax.fori_loop(0, CW, _elu, 0)
            pltpu.sync_copy(rows_v, out_hbm.at[c * 2 + p, pl.ds(r0, CW), :])


def kernel(node_features, edge_index, W, a_w):
    a1 = a_w[0, :DOUT]
    a2 = a_w[0, DOUT:]
    wt = W.T
    w_ext = jnp.concatenate(
        [wt, (wt @ a1)[:, None], (wt @ a2)[:, None],
         jnp.zeros((DIN, 126), jnp.float32)], axis=1)

    out1 = _tc_matmul(node_features, w_ext)
    h = out1[:, :DOUT]
    s1 = jnp.pad(out1[:, DOUT], (0, NPAD - N))
    s2 = jnp.pad(out1[:, DOUT + 1], (0, NPAD - N))

    hs = jnp.concatenate([h[:, q * QW:(q + 1) * QW] for q in range(4)], axis=0)

    src = jnp.pad(edge_index[0], (0, EPAD - E)).reshape(16 * CHUNKS, CW)
    tgt = jnp.pad(edge_index[1], (0, EPAD - E)).reshape(16 * CHUNKS, CW)

    out = _sc_edges(hs, s1, s2, src, tgt)
    return jnp.concatenate([out[q, :N] for q in range(4)], axis=1)


# speedup vs baseline: 5.7899x; 1.5343x over previous
import functools

import jax
import jax.numpy as jnp
from jax import lax
from jax.experimental import pallas as pl
from jax.experimental.pallas import tpu as pltpu
from jax.experimental.pallas import tpu_sc as plsc

N = 10000
E = 160000
DIN = 256
DOUT = 256
ALPHA = 0.2

NPAD = 10240
EPAD = 163840
CHUNKS = 80
CW = 128
ROWS_PT = NPAD // 16
HALF = 128
QW = 32
NQ = DOUT // QW


def _mm_body(x_ref, w_ref, o_ref):
    o_ref[...] = jnp.dot(x_ref[...], w_ref[...],
                         preferred_element_type=jnp.float32)


def _tc_matmul(x, w_ext):
    bn = 2000
    return pl.pallas_call(
        _mm_body,
        grid=(N // bn,),
        in_specs=[
            pl.BlockSpec((bn, DIN), lambda i: (i, 0)),
            pl.BlockSpec((DIN, 384), lambda i: (0, 0)),
        ],
        out_specs=pl.BlockSpec((bn, 384), lambda i: (i, 0)),
        out_shape=jax.ShapeDtypeStruct((N, 384), jnp.float32),
    )(x, w_ext)


_sc_mesh = plsc.VectorSubcoreMesh(core_axis_name="c", subcore_axis_name="s")


@functools.partial(
    pl.kernel,
    mesh=_sc_mesh,
    compiler_params=pltpu.CompilerParams(needs_layout_passes=False,
                                         use_tc_tiling_on_sc=False),
    out_type=jax.ShapeDtypeStruct((NQ, NPAD, QW), jnp.float32),
    scratch_types=[
        pltpu.VMEM((CHUNKS, CW), jnp.int32),
        pltpu.VMEM((CHUNKS, CW), jnp.int32),
        pltpu.VMEM((CHUNKS, CW), jnp.int32),
        pltpu.VMEM((NPAD,), jnp.float32),
        pltpu.VMEM((NPAD,), jnp.float32),
        pltpu.VMEM((NPAD,), jnp.float32),
        pltpu.VMEM((CHUNKS, CW), jnp.float32),
        pltpu.VMEM((2, CW, QW), jnp.float32),
        pltpu.VMEM((2, CW, QW), jnp.float32),
        pltpu.VMEM_SHARED((NPAD,), jnp.float32),
        pltpu.VMEM_SHARED((NPAD, QW), jnp.float32),
        pltpu.SemaphoreType.DMA((2,)),
        pltpu.SemaphoreType.DMA((2,)),
    ],
)
def _sc_edges(hs_hbm, s1_hbm, s2_hbm, src_hbm, tgt_hbm, out_hbm,
              src_v, tgt_v, srcadj_v, s1_v, s2_v, sums_v, att_v,
              grows_v, srows_v, sums_sh, acc_sh, gsem, ssem):
    t = lax.axis_index("s")
    c = lax.axis_index("c")
    zero16 = jnp.zeros((16,), jnp.float32)
    iota16 = lax.iota(jnp.int32, 16)

    pltpu.sync_copy(src_hbm.at[pl.ds(t * CHUNKS, CHUNKS)], src_v)
    pltpu.sync_copy(tgt_hbm.at[pl.ds(t * CHUNKS, CHUNKS)], tgt_v)
    pltpu.sync_copy(s1_hbm, s1_v)
    pltpu.sync_copy(s2_hbm, s2_v)

    def _zsum(i, _):
        sums_v[pl.ds(i * 16, 16)] = zero16
        return 0
    lax.fori_loop(0, ROWS_PT // 16, _zsum, 0)
    pltpu.sync_copy(sums_v.at[pl.ds(0, ROWS_PT)],
                    sums_sh.at[pl.ds(t * ROWS_PT, ROWS_PT)])

    def _edges(j, _):
        for k in range(8):
            sl = pl.ds(k * 16, 16)
            sv = src_v[j, sl]
            tv = tgt_v[j, sl]
            v1 = plsc.load_gather(s1_v, [sv])
            v2 = plsc.load_gather(s2_v, [tv])
            lg = v1 + v2
            lg = jnp.where(lg > 0.0, lg, lg * ALPHA)
            ee = jnp.exp(lg)
            gid = t * (CHUNKS * CW) + j * CW + k * 16 + iota16
            ee = jnp.where(gid < E, ee, 0.0)
            att_v[j, sl] = ee
        return 0
    lax.fori_loop(0, CHUNKS, _edges, 0)

    plsc.subcore_barrier()

    def _ssum(j, _):
        pltpu.sync_copy(att_v.at[j], sums_sh.at[tgt_v.at[j]], add=True)
        return 0
    lax.fori_loop(0, CHUNKS, _ssum, 0)

    plsc.subcore_barrier()

    pltpu.sync_copy(sums_sh, sums_v)

    def _att(j, _):
        for k in range(8):
            sl = pl.ds(k * 16, 16)
            tv = tgt_v[j, sl]
            s = plsc.load_gather(sums_v, [tv])
            att_v[j, sl] = att_v[j, sl] / (s + 1e-10)
        return 0
    lax.fori_loop(0, CHUNKS, _att, 0)

    for p in range(HALF // QW):
        def _adj(j, _):
            for k in range(8):
                sl = pl.ds(k * 16, 16)
                srcadj_v[j, sl] = src_v[j, sl] + (c * (HALF // QW) + p) * N
            return 0
        lax.fori_loop(0, CHUNKS, _adj, 0)

        def _zrow(r, _):
            for k in range(QW // 16):
                srows_v[0, r, pl.ds(k * 16, 16)] = zero16
            return 0
        lax.fori_loop(0, CW, _zrow, 0)
        for b in range(ROWS_PT // CW):
            pltpu.sync_copy(srows_v.at[0],
                            acc_sh.at[pl.ds(t * ROWS_PT + b * CW, CW)])

        plsc.subcore_barrier()

        def _g_start(j, b):
            pltpu.async_copy(hs_hbm.at[srcadj_v.at[j]], grows_v.at[b],
                             gsem.at[b])

        def _g_wait(j, b):
            pltpu.make_async_copy(hs_hbm.at[srcadj_v.at[j]], grows_v.at[b],
                                  gsem.at[b]).wait()

        def _s_start(j, b):
            pltpu.async_copy(srows_v.at[b], acc_sh.at[tgt_v.at[j]],
                             ssem.at[b], add=True)

        def _s_wait(j, b):
            pltpu.make_async_copy(srows_v.at[b], acc_sh.at[tgt_v.at[j]],
                                  ssem.at[b]).wait()

        _g_start(0, 0)
        _g_start(1, 1)

        def _ring(g, _):
            for b in range(2):
                j = g * 2 + b
                _g_wait(j, b)

                @pl.when(g > 0)
                def _():
                    _s_wait(j - 2, b)

                def _scale(gg, _):
                    av = att_v[j, pl.ds(gg * 16, 16)]
                    for ri in range(16):
                        a = av[ri]
                        r = gg * 16 + ri
                        for k in range(QW // 16):
                            sl = pl.ds(k * 16, 16)
                            srows_v[b, r, sl] = grows_v[b, r, sl] * a
                    return 0
                lax.fori_loop(0, CW // 16, _scale, 0)
                _s_start(j, b)

                @pl.when(g < CHUNKS // 2 - 1)
                def _():
                    _g_start(j + 2, b)
            return 0
        lax.fori_loop(0, CHUNKS // 2, _ring, 0)

        _s_wait(CHUNKS - 2, 0)
        _s_wait(CHUNKS - 1, 1)

        plsc.subcore_barrier()

        for b in range(ROWS_PT // CW):
            r0 = t * ROWS_PT + b * CW
            pltpu.sync_copy(acc_sh.at[pl.ds(r0, CW)], srows_v.at[0])

            def _elu(r, _):
                for k in range(QW // 16):
                    sl = pl.ds(k * 16, 16)
                    v = srows_v[0, r, sl]
                    srows_v[0, r, sl] = jnp.where(v > 0.0, v,
                                                  jnp.exp(v) - 1.0)
                return 0
            lax.fori_loop(0, CW, _elu, 0)
            pltpu.sync_copy(srows_v.at[0],
                            out_hbm.at[c * (HALF // QW) + p, pl.ds(r0, CW), :])


def kernel(node_features, edge_index, W, a_w):
    a1 = a_w[0, :DOUT]
    a2 = a_w[0, DOUT:]
    wt = W.T
    w_ext = jnp.concatenate(
        [wt, (wt @ a1)[:, None], (wt @ a2)[:, None],
         jnp.zeros((DIN, 126), jnp.float32)], axis=1)

    out1 = _tc_matmul(node_features, w_ext)
    h = out1[:, :DOUT]
    s1 = jnp.pad(out1[:, DOUT], (0, NPAD - N))
    s2 = jnp.pad(out1[:, DOUT + 1], (0, NPAD - N))

    hs = jnp.concatenate([h[:, q * QW:(q + 1) * QW] for q in range(NQ)], axis=0)

    src = jnp.pad(edge_index[0], (0, EPAD - E)).reshape(16 * CHUNKS, CW)
    tgt = jnp.pad(edge_index[1], (0, EPAD - E)).reshape(16 * CHUNKS, CW)

    out = _sc_edges(hs, s1, s2, src, tgt)
    return jnp.concatenate([out[q, :N] for q in range(NQ)], axis=1)


# speedup vs baseline: 6.7894x; 1.1726x over previous
import functools

import jax
import jax.numpy as jnp
from jax import lax
from jax.experimental import pallas as pl
from jax.experimental.pallas import tpu as pltpu
from jax.experimental.pallas import tpu_sc as plsc

N = 10000
E = 160000
DIN = 256
DOUT = 256
ALPHA = 0.2

NPAD = 10240
EPAD = 163840
CHUNKS = 80
CW = 128
ROWS_PT = NPAD // 16
HALF = 128
QW = 32
NQ = DOUT // QW


def _mm_body(x_ref, w_ref, hs_ref, s_ref, acc_ref):
    acc_ref[...] = jnp.dot(x_ref[...], w_ref[...],
                           preferred_element_type=jnp.float32)
    for q in range(NQ):
        hs_ref[q] = acc_ref[:, q * QW:(q + 1) * QW]
    s_ref[...] = acc_ref[:, DOUT:DOUT + 128]


def _tc_matmul(x, w_ext):
    bn = 2000
    return pl.pallas_call(
        _mm_body,
        grid=(N // bn,),
        in_specs=[
            pl.BlockSpec((bn, DIN), lambda i: (i, 0)),
            pl.BlockSpec((DIN, 384), lambda i: (0, 0)),
        ],
        out_specs=[
            pl.BlockSpec((NQ, bn, QW), lambda i: (0, i, 0)),
            pl.BlockSpec((bn, 128), lambda i: (i, 0)),
        ],
        out_shape=[
            jax.ShapeDtypeStruct((NQ, N, QW), jnp.float32),
            jax.ShapeDtypeStruct((N, 128), jnp.float32),
        ],
        scratch_shapes=[pltpu.VMEM((bn, 384), jnp.float32)],
    )(x, w_ext)


_sc_mesh = plsc.VectorSubcoreMesh(core_axis_name="c", subcore_axis_name="s")


@functools.partial(
    pl.kernel,
    mesh=_sc_mesh,
    compiler_params=pltpu.CompilerParams(needs_layout_passes=False,
                                         use_tc_tiling_on_sc=False),
    out_type=jax.ShapeDtypeStruct((NPAD, DOUT), jnp.float32),
    scratch_types=[
        pltpu.VMEM((CHUNKS, CW), jnp.int32),
        pltpu.VMEM((CHUNKS, CW), jnp.int32),
        pltpu.VMEM((CHUNKS, CW), jnp.int32),
        pltpu.VMEM((NPAD,), jnp.float32),
        pltpu.VMEM((NPAD,), jnp.float32),
        pltpu.VMEM((NPAD,), jnp.float32),
        pltpu.VMEM((CHUNKS, CW), jnp.float32),
        pltpu.VMEM((2, CW, QW), jnp.float32),
        pltpu.VMEM((2, CW, QW), jnp.float32),
        pltpu.VMEM_SHARED((NPAD,), jnp.float32),
        pltpu.VMEM_SHARED((NPAD, QW), jnp.float32),
        pltpu.SemaphoreType.DMA((2,)),
        pltpu.SemaphoreType.DMA((2,)),
    ],
)
def _sc_edges(hs_hbm, s1_hbm, s2_hbm, src_hbm, tgt_hbm, out_hbm,
              src_v, tgt_v, srcadj_v, s1_v, s2_v, sums_v, att_v,
              grows_v, srows_v, sums_sh, acc_sh, gsem, ssem):
    t = lax.axis_index("s")
    c = lax.axis_index("c")
    zero16 = jnp.zeros((16,), jnp.float32)
    iota16 = lax.iota(jnp.int32, 16)

    pltpu.sync_copy(src_hbm.at[pl.ds(t * CHUNKS, CHUNKS)], src_v)
    pltpu.sync_copy(tgt_hbm.at[pl.ds(t * CHUNKS, CHUNKS)], tgt_v)
    pltpu.sync_copy(s1_hbm, s1_v)
    pltpu.sync_copy(s2_hbm, s2_v)

    def _zsum(i, _):
        sums_v[pl.ds(i * 16, 16)] = zero16
        return 0
    lax.fori_loop(0, ROWS_PT // 16, _zsum, 0)
    pltpu.sync_copy(sums_v.at[pl.ds(0, ROWS_PT)],
                    sums_sh.at[pl.ds(t * ROWS_PT, ROWS_PT)])

    def _edges(j, _):
        for k in range(8):
            sl = pl.ds(k * 16, 16)
            sv = src_v[j, sl]
            tv = tgt_v[j, sl]
            v1 = plsc.load_gather(s1_v, [sv])
            v2 = plsc.load_gather(s2_v, [tv])
            lg = v1 + v2
            lg = jnp.where(lg > 0.0, lg, lg * ALPHA)
            ee = jnp.exp(lg)
            gid = t * (CHUNKS * CW) + j * CW + k * 16 + iota16
            ee = jnp.where(gid < E, ee, 0.0)
            att_v[j, sl] = ee
        return 0
    lax.fori_loop(0, CHUNKS, _edges, 0)

    plsc.subcore_barrier()

    def _ssum(j, _):
        pltpu.sync_copy(att_v.at[j], sums_sh.at[tgt_v.at[j]], add=True)
        return 0
    lax.fori_loop(0, CHUNKS, _ssum, 0)

    plsc.subcore_barrier()

    pltpu.sync_copy(sums_sh, sums_v)

    def _att(j, _):
        for k in range(8):
            sl = pl.ds(k * 16, 16)
            tv = tgt_v[j, sl]
            s = plsc.load_gather(sums_v, [tv])
            att_v[j, sl] = att_v[j, sl] / (s + 1e-10)
        return 0
    lax.fori_loop(0, CHUNKS, _att, 0)

    for p in range(HALF // QW):
        def _adj(j, _):
            for k in range(8):
                sl = pl.ds(k * 16, 16)
                srcadj_v[j, sl] = src_v[j, sl] + (c * (HALF // QW) + p) * N
            return 0
        lax.fori_loop(0, CHUNKS, _adj, 0)

        def _zrow(r, _):
            for k in range(QW // 16):
                srows_v[0, r, pl.ds(k * 16, 16)] = zero16
            return 0
        lax.fori_loop(0, CW, _zrow, 0)
        for b in range(ROWS_PT // CW):
            pltpu.sync_copy(srows_v.at[0],
                            acc_sh.at[pl.ds(t * ROWS_PT + b * CW, CW)])

        plsc.subcore_barrier()

        def _g_start(j, b):
            pltpu.async_copy(hs_hbm.at[srcadj_v.at[j]], grows_v.at[b],
                             gsem.at[b])

        def _g_wait(j, b):
            pltpu.make_async_copy(hs_hbm.at[srcadj_v.at[j]], grows_v.at[b],
                                  gsem.at[b]).wait()

        def _s_start(j, b):
            pltpu.async_copy(srows_v.at[b], acc_sh.at[tgt_v.at[j]],
                             ssem.at[b], add=True)

        def _s_wait(j, b):
            pltpu.make_async_copy(srows_v.at[b], acc_sh.at[tgt_v.at[j]],
                                  ssem.at[b]).wait()

        _g_start(0, 0)
        _g_start(1, 1)

        def _ring(g, _):
            for b in range(2):
                j = g * 2 + b
                _g_wait(j, b)

                @pl.when(g > 0)
                def _():
                    _s_wait(j - 2, b)

                def _scale(gg, _):
                    av = att_v[j, pl.ds(gg * 16, 16)]
                    for ri in range(16):
                        a = av[ri]
                        r = gg * 16 + ri
                        for k in range(QW // 16):
                            sl = pl.ds(k * 16, 16)
                            srows_v[b, r, sl] = grows_v[b, r, sl] * a
                    return 0
                lax.fori_loop(0, CW // 16, _scale, 0)
                _s_start(j, b)

                @pl.when(g < CHUNKS // 2 - 1)
                def _():
                    _g_start(j + 2, b)
            return 0
        lax.fori_loop(0, CHUNKS // 2, _ring, 0)

        _s_wait(CHUNKS - 2, 0)
        _s_wait(CHUNKS - 1, 1)

        plsc.subcore_barrier()

        for b in range(ROWS_PT // CW):
            r0 = t * ROWS_PT + b * CW
            pltpu.sync_copy(acc_sh.at[pl.ds(r0, CW)], srows_v.at[0])

            def _elu(r, _):
                for k in range(QW // 16):
                    sl = pl.ds(k * 16, 16)
                    v = srows_v[0, r, sl]
                    srows_v[0, r, sl] = jnp.where(v > 0.0, v,
                                                  jnp.exp(v) - 1.0)
                return 0
            lax.fori_loop(0, CW, _elu, 0)
            pltpu.sync_copy(
                srows_v.at[0],
                out_hbm.at[pl.ds(r0, CW),
                           pl.ds((c * (HALF // QW) + p) * QW, QW)])


def kernel(node_features, edge_index, W, a_w):
    a1 = a_w[0, :DOUT]
    a2 = a_w[0, DOUT:]
    wt = W.T
    w_ext = jnp.concatenate(
        [wt, (wt @ a1)[:, None], (wt @ a2)[:, None],
         jnp.zeros((DIN, 126), jnp.float32)], axis=1)

    hs3, scores = _tc_matmul(node_features, w_ext)
    hs = hs3.reshape(NQ * N, QW)
    s1 = jnp.pad(scores[:, 0], (0, NPAD - N))
    s2 = jnp.pad(scores[:, 1], (0, NPAD - N))

    src = jnp.pad(edge_index[0], (0, EPAD - E)).reshape(16 * CHUNKS, CW)
    tgt = jnp.pad(edge_index[1], (0, EPAD - E)).reshape(16 * CHUNKS, CW)

    out = _sc_edges(hs, s1, s2, src, tgt)
    return out[:N]
